# baseline (speedup 1.0000x reference)
import functools

import jax
import jax.numpy as jnp
from jax import lax
from jax.experimental import pallas as pl
from jax.experimental.pallas import tpu as pltpu
from jax.experimental.pallas import tpu_sc as plsc

_D = 256
_K = 16
_NC = 2
_NS = 16
_NW = _NC * _NS
_BP = 10240
_BW = _BP // _NW
_CH = 8
_NCHUNK = _BW // _CH
_NG = _D // 16


def _sc_body(nodes_hbm, nidx_hbm, emb_hbm, self_hbm, neigh_hbm,
             nodes_v, nids2_v, nidsf_v, rows0_v, rows1_v, obuf0_v, obuf1_v,
             selfbuf_v, sem_n, sem_s, sem_g0, sem_g1, sem_o0, sem_o1):
    c = lax.axis_index("c")
    s = lax.axis_index("s")
    wid = c * _NS + s
    base = wid * _BW

    pltpu.sync_copy(nodes_hbm.at[pl.ds(base, _BW)], nodes_v)

    for k in range(4):
        pltpu.async_copy(
            nidx_hbm.at[nodes_v.at[pl.ds(k * 80, 80)]], nids2_v, sem_n
        ).wait()

        def _flat(i, _):
            nidsf_v[pl.ds((k * 80 + i) * _K, _K)] = nids2_v[i, pl.ds(0, _K)]
            return 0
        lax.fori_loop(0, 80, _flat, 0)

    for k in range(4):
        pltpu.async_copy(
            emb_hbm.at[nodes_v.at[pl.ds(k * 80, 80)]], selfbuf_v, sem_s
        ).wait()
        pltpu.sync_copy(selfbuf_v, self_hbm.at[pl.ds(base + k * 80, 80)])

    rows = (rows0_v, rows1_v)
    obufs = (obuf0_v, obuf1_v)
    sems = (sem_g0, sem_g1)
    sems_o = (sem_o0, sem_o1)

    def _fire(ci, b):
        return pltpu.async_copy(
            emb_hbm.at[nidsf_v.at[pl.ds(ci * (_CH * _K), _CH * _K)]],
            rows[b], sems[b])

    _fire(0, 0)
    _fire(1, 1)

    def _pair(p, _):
        for b in range(2):
            ci = p * 2 + b
            pltpu.make_async_copy(
                emb_hbm.at[nidsf_v.at[pl.ds(ci * (_CH * _K), _CH * _K)]],
                rows[b], sems[b]).wait()
            @pl.when(p > 0)
            def _():
                pltpu.make_async_copy(
                    obufs[b], neigh_hbm.at[pl.ds(base, _CH)], sems_o[b]
                ).wait()

            def _node(j, _):
                r0 = j * _K
                for g in range(_NG):
                    acc = rows[b][r0, pl.ds(g * 16, 16)]
                    for r in range(1, _K):
                        acc = acc + rows[b][r0 + r, pl.ds(g * 16, 16)]
                    obufs[b][j, pl.ds(g * 16, 16)] = acc
                return 0
            lax.fori_loop(0, _CH, _node, 0)

            pltpu.async_copy(
                obufs[b], neigh_hbm.at[pl.ds(base + ci * _CH, _CH)],
                sems_o[b])

            @pl.when(ci + 2 < _NCHUNK)
            def _():
                _fire(ci + 2, b)
        return 0
    lax.fori_loop(0, _NCHUNK // 2, _pair, 0)

    for b in range(2):
        pltpu.make_async_copy(
            obufs[b], neigh_hbm.at[pl.ds(base, _CH)], sems_o[b]).wait()


def _sc_gather(nodes_p, nidx_p, emb):
    mesh = plsc.VectorSubcoreMesh(core_axis_name="c", subcore_axis_name="s")
    f = pl.kernel(
        _sc_body,
        out_type=(
            jax.ShapeDtypeStruct((_BP, _D), jnp.float32),
            jax.ShapeDtypeStruct((_BP, _D), jnp.float32),
        ),
        mesh=mesh,
        scratch_types=[
            pltpu.VMEM((_BW,), jnp.int32),
            pltpu.VMEM((80, 128), jnp.int32),
            pltpu.VMEM((_BW * _K,), jnp.int32),
            pltpu.VMEM((_CH * _K, _D), jnp.float32),
            pltpu.VMEM((_CH * _K, _D), jnp.float32),
            pltpu.VMEM((_CH, _D), jnp.float32),
            pltpu.VMEM((_CH, _D), jnp.float32),
            pltpu.VMEM((80, _D), jnp.float32),
            pltpu.SemaphoreType.DMA,
            pltpu.SemaphoreType.DMA,
            pltpu.SemaphoreType.DMA,
            pltpu.SemaphoreType.DMA,
            pltpu.SemaphoreType.DMA,
            pltpu.SemaphoreType.DMA,
        ],
    )
    return f(nodes_p, nidx_p, emb)


def _tc_body(w1_ref, w2_ref, xs_ref, xn_ref, o_ref):
    a = lax.dot_general(w1_ref[...], xs_ref[...],
                        (((1,), (1,)), ((), ())),
                        preferred_element_type=jnp.float32)
    b = lax.dot_general(w2_ref[...], xn_ref[...],
                        (((1,), (1,)), ((), ())),
                        preferred_element_type=jnp.float32)
    o_ref[...] = jnp.maximum(a + b, 0.0)


def _tc_combine(w1, w2, xs, xn):
    blk = 2048
    grid = _BP // blk
    return pl.pallas_call(
        _tc_body,
        grid=(grid,),
        in_specs=[
            pl.BlockSpec((_D, _D), lambda i: (0, 0)),
            pl.BlockSpec((_D, _D), lambda i: (0, 0)),
            pl.BlockSpec((blk, _D), lambda i: (i, 0)),
            pl.BlockSpec((blk, _D), lambda i: (i, 0)),
        ],
        out_specs=pl.BlockSpec((_D, blk), lambda i: (0, i)),
        out_shape=jax.ShapeDtypeStruct((_D, _BP), jnp.float32),
    )(w1, w2, xs, xn)


def kernel(nodes, emb, neigh_idx, W):
    B = nodes.shape[0]
    nodes32 = nodes.astype(jnp.int32)
    nidx32 = neigh_idx.astype(jnp.int32)
    nidx_p = jnp.pad(nidx32, ((0, 0), (0, 128 - _K)))
    nodes_p = jnp.zeros((_BP,), jnp.int32).at[:B].set(nodes32)
    self_f, neigh_s = _sc_gather(nodes_p, nidx_p, emb)
    w1 = W[:, :_D]
    w2 = W[:, _D:] * (1.0 / _K)
    out_p = _tc_combine(w1, w2, self_f, neigh_s)
    return out_p[:, :B]


# speedup vs baseline: 1.1020x; 1.1020x over previous
import functools

import jax
import jax.numpy as jnp
from jax import lax
from jax.experimental import pallas as pl
from jax.experimental.pallas import tpu as pltpu
from jax.experimental.pallas import tpu_sc as plsc

_D = 256
_K = 16
_NC = 2
_NS = 16
_NW = _NC * _NS
_BP = 10240
_BW = _BP // _NW
_CH = 8
_NCHUNK = _BW // _CH
_NG = _D // 16


def _sc_body(nodes_hbm, nidx_hbm, emb_hbm, self_hbm, neigh_hbm,
             nodes_v, nids2_v, nidsf_v, rows0_v, rows1_v, obuf0_v, obuf1_v,
             selfbuf_v, sem_n, sem_s, sem_g0, sem_g1, sem_o0, sem_o1):
    c = lax.axis_index("c")
    s = lax.axis_index("s")
    wid = c * _NS + s
    base = wid * _BW

    pltpu.sync_copy(nodes_hbm.at[pl.ds(base, _BW)], nodes_v)

    for k in range(4):
        pltpu.async_copy(
            nidx_hbm.at[nodes_v.at[pl.ds(k * 80, 80)]], nids2_v, sem_n
        ).wait()

        def _flat(i, _):
            nidsf_v[pl.ds((k * 80 + i) * _K, _K)] = nids2_v[i, pl.ds(0, _K)]
            return 0
        lax.fori_loop(0, 80, _flat, 0)

    for k in range(4):
        pltpu.async_copy(
            emb_hbm.at[nodes_v.at[pl.ds(k * 80, 80)]], selfbuf_v, sem_s
        ).wait()
        pltpu.sync_copy(selfbuf_v, self_hbm.at[pl.ds(base + k * 80, 80)])

    rows = (rows0_v, rows1_v)
    obufs = (obuf0_v, obuf1_v)
    sems = (sem_g0, sem_g1)
    sems_o = (sem_o0, sem_o1)

    def _fire(ci, b):
        return pltpu.async_copy(
            emb_hbm.at[nidsf_v.at[pl.ds(ci * (_CH * _K), _CH * _K)]],
            rows[b], sems[b])

    _fire(0, 0)
    _fire(1, 1)

    def _pair(p, _):
        for b in range(2):
            ci = p * 2 + b
            pltpu.make_async_copy(
                emb_hbm.at[nidsf_v.at[pl.ds(ci * (_CH * _K), _CH * _K)]],
                rows[b], sems[b]).wait()
            @pl.when(p > 0)
            def _():
                pltpu.make_async_copy(
                    obufs[b], neigh_hbm.at[pl.ds(base, _CH)], sems_o[b]
                ).wait()

            def _node(j, _):
                r0 = j * _K
                for g in range(_NG):
                    v = [rows[b][r0 + r, pl.ds(g * 16, 16)]
                         for r in range(_K)]
                    while len(v) > 1:
                        v = [v[2 * i] + v[2 * i + 1]
                             for i in range(len(v) // 2)]
                    obufs[b][j, pl.ds(g * 16, 16)] = v[0]
                return 0
            lax.fori_loop(0, _CH, _node, 0)

            pltpu.async_copy(
                obufs[b], neigh_hbm.at[pl.ds(base + ci * _CH, _CH)],
                sems_o[b])

            @pl.when(ci + 2 < _NCHUNK)
            def _():
                _fire(ci + 2, b)
        return 0
    lax.fori_loop(0, _NCHUNK // 2, _pair, 0)

    for b in range(2):
        pltpu.make_async_copy(
            obufs[b], neigh_hbm.at[pl.ds(base, _CH)], sems_o[b]).wait()


def _sc_gather(nodes_p, nidx_p, emb):
    mesh = plsc.VectorSubcoreMesh(core_axis_name="c", subcore_axis_name="s")
    f = pl.kernel(
        _sc_body,
        out_type=(
            jax.ShapeDtypeStruct((_BP, _D), jnp.float32),
            jax.ShapeDtypeStruct((_BP, _D), jnp.float32),
        ),
        mesh=mesh,
        scratch_types=[
            pltpu.VMEM((_BW,), jnp.int32),
            pltpu.VMEM((80, 128), jnp.int32),
            pltpu.VMEM((_BW * _K,), jnp.int32),
            pltpu.VMEM((_CH * _K, _D), jnp.float32),
            pltpu.VMEM((_CH * _K, _D), jnp.float32),
            pltpu.VMEM((_CH, _D), jnp.float32),
            pltpu.VMEM((_CH, _D), jnp.float32),
            pltpu.VMEM((80, _D), jnp.float32),
            pltpu.SemaphoreType.DMA,
            pltpu.SemaphoreType.DMA,
            pltpu.SemaphoreType.DMA,
            pltpu.SemaphoreType.DMA,
            pltpu.SemaphoreType.DMA,
            pltpu.SemaphoreType.DMA,
        ],
    )
    return f(nodes_p, nidx_p, emb)


def _tc_body(w1_ref, w2_ref, xs_ref, xn_ref, o_ref):
    a = lax.dot_general(w1_ref[...], xs_ref[...],
                        (((1,), (1,)), ((), ())),
                        preferred_element_type=jnp.float32)
    b = lax.dot_general(w2_ref[...], xn_ref[...],
                        (((1,), (1,)), ((), ())),
                        preferred_element_type=jnp.float32)
    o_ref[...] = jnp.maximum(a + b, 0.0)


def _tc_combine(w1, w2, xs, xn, n_out):
    blk = 2048
    grid = _BP // blk
    return pl.pallas_call(
        _tc_body,
        grid=(grid,),
        in_specs=[
            pl.BlockSpec((_D, _D), lambda i: (0, 0)),
            pl.BlockSpec((_D, _D), lambda i: (0, 0)),
            pl.BlockSpec((blk, _D), lambda i: (i, 0)),
            pl.BlockSpec((blk, _D), lambda i: (i, 0)),
        ],
        out_specs=pl.BlockSpec((_D, blk), lambda i: (0, i)),
        out_shape=jax.ShapeDtypeStruct((_D, n_out), jnp.float32),
    )(w1, w2, xs, xn)


def kernel(nodes, emb, neigh_idx, W):
    B = nodes.shape[0]
    nodes32 = nodes.astype(jnp.int32)
    nidx32 = neigh_idx.astype(jnp.int32)
    nidx_p = jnp.pad(nidx32, ((0, 0), (0, 128 - _K)))
    nodes_p = jnp.zeros((_BP,), jnp.int32).at[:B].set(nodes32)
    self_f, neigh_s = _sc_gather(nodes_p, nidx_p, emb)
    w1 = W[:, :_D]
    w2 = W[:, _D:] * (1.0 / _K)
    return _tc_combine(w1, w2, self_f, neigh_s, B)


# speedup vs baseline: 1.1024x; 1.0004x over previous
import functools

import jax
import jax.numpy as jnp
from jax import lax
from jax.experimental import pallas as pl
from jax.experimental.pallas import tpu as pltpu
from jax.experimental.pallas import tpu_sc as plsc

_D = 256
_K = 16
_NC = 2
_NS = 16
_NW = _NC * _NS
_BP = 10240
_BW = _BP // _NW
_CH = 8
_NCHUNK = _BW // _CH
_NG = _D // 16


def _sc_body(nodes_hbm, nidx_hbm, emb_hbm, self_hbm, neigh_hbm,
             nodes_v, nids2_v, nidsf_v, rows0_v, rows1_v, obuf0_v, obuf1_v,
             selfbuf_v, sem_n, sem_s, sem_g0, sem_g1, sem_o0, sem_o1):
    c = lax.axis_index("c")
    s = lax.axis_index("s")
    wid = c * _NS + s
    base = wid * _BW

    with jax.named_scope("sc_nids"):
        pltpu.sync_copy(nodes_hbm.at[pl.ds(base, _BW)], nodes_v)
        for k in range(4):
            pltpu.async_copy(
                nidx_hbm.at[nodes_v.at[pl.ds(k * 80, 80)]], nids2_v, sem_n
            ).wait()

            def _flat(i, _):
                nidsf_v[pl.ds((k * 80 + i) * _K, _K)] = (
                    nids2_v[i, pl.ds(0, _K)])
                return 0
            lax.fori_loop(0, 80, _flat, 0)

    with jax.named_scope("sc_self"):
        for k in range(4):
            pltpu.async_copy(
                emb_hbm.at[nodes_v.at[pl.ds(k * 80, 80)]], selfbuf_v, sem_s
            ).wait()
            pltpu.sync_copy(selfbuf_v, self_hbm.at[pl.ds(base + k * 80, 80)])

    rows = (rows0_v, rows1_v)
    obufs = (obuf0_v, obuf1_v)
    sems = (sem_g0, sem_g1)
    sems_o = (sem_o0, sem_o1)

    def _fire(ci, b):
        return pltpu.async_copy(
            emb_hbm.at[nidsf_v.at[pl.ds(ci * (_CH * _K), _CH * _K)]],
            rows[b], sems[b])

    with jax.named_scope("sc_prime"):
        _fire(0, 0)
        _fire(1, 1)

    def _pair(p, _):
        for b in range(2):
            ci = p * 2 + b
            pltpu.make_async_copy(
                emb_hbm.at[nidsf_v.at[pl.ds(ci * (_CH * _K), _CH * _K)]],
                rows[b], sems[b]).wait()
            @pl.when(p > 0)
            def _():
                pltpu.make_async_copy(
                    obufs[b], neigh_hbm.at[pl.ds(base, _CH)], sems_o[b]
                ).wait()

            def _node(j, _):
                r0 = j * _K
                for g in range(_NG):
                    v = [rows[b][r0 + r, pl.ds(g * 16, 16)]
                         for r in range(_K)]
                    while len(v) > 1:
                        v = [v[2 * i] + v[2 * i + 1]
                             for i in range(len(v) // 2)]
                    obufs[b][j, pl.ds(g * 16, 16)] = v[0]
                return 0
            lax.fori_loop(0, _CH, _node, 0)

            pltpu.async_copy(
                obufs[b], neigh_hbm.at[pl.ds(base + ci * _CH, _CH)],
                sems_o[b])

            @pl.when(ci + 2 < _NCHUNK)
            def _():
                _fire(ci + 2, b)
        return 0
    with jax.named_scope("sc_main"):
        lax.fori_loop(0, _NCHUNK // 2, _pair, 0)

    with jax.named_scope("sc_drain"):
        for b in range(2):
            pltpu.make_async_copy(
                obufs[b], neigh_hbm.at[pl.ds(base, _CH)], sems_o[b]).wait()


def _sc_gather(nodes_p, nidx_p, emb):
    mesh = plsc.VectorSubcoreMesh(core_axis_name="c", subcore_axis_name="s")
    f = pl.kernel(
        _sc_body,
        out_type=(
            jax.ShapeDtypeStruct((_BP, _D), jnp.float32),
            jax.ShapeDtypeStruct((_BP, _D), jnp.float32),
        ),
        mesh=mesh,
        scratch_types=[
            pltpu.VMEM((_BW,), jnp.int32),
            pltpu.VMEM((80, 128), jnp.int32),
            pltpu.VMEM((_BW * _K,), jnp.int32),
            pltpu.VMEM((_CH * _K, _D), jnp.float32),
            pltpu.VMEM((_CH * _K, _D), jnp.float32),
            pltpu.VMEM((_CH, _D), jnp.float32),
            pltpu.VMEM((_CH, _D), jnp.float32),
            pltpu.VMEM((80, _D), jnp.float32),
            pltpu.SemaphoreType.DMA,
            pltpu.SemaphoreType.DMA,
            pltpu.SemaphoreType.DMA,
            pltpu.SemaphoreType.DMA,
            pltpu.SemaphoreType.DMA,
            pltpu.SemaphoreType.DMA,
        ],
    )
    return f(nodes_p, nidx_p, emb)


def _tc_body(w1_ref, w2_ref, xs_ref, xn_ref, o_ref):
    a = lax.dot_general(w1_ref[...], xs_ref[...],
                        (((1,), (1,)), ((), ())),
                        preferred_element_type=jnp.float32)
    b = lax.dot_general(w2_ref[...], xn_ref[...],
                        (((1,), (1,)), ((), ())),
                        preferred_element_type=jnp.float32)
    o_ref[...] = jnp.maximum(a + b, 0.0)


def _tc_combine(w1, w2, xs, xn, n_out):
    blk = 2048
    grid = _BP // blk
    return pl.pallas_call(
        _tc_body,
        grid=(grid,),
        in_specs=[
            pl.BlockSpec((_D, _D), lambda i: (0, 0)),
            pl.BlockSpec((_D, _D), lambda i: (0, 0)),
            pl.BlockSpec((blk, _D), lambda i: (i, 0)),
            pl.BlockSpec((blk, _D), lambda i: (i, 0)),
        ],
        out_specs=pl.BlockSpec((_D, blk), lambda i: (0, i)),
        out_shape=jax.ShapeDtypeStruct((_D, n_out), jnp.float32),
    )(w1, w2, xs, xn)


def kernel(nodes, emb, neigh_idx, W):
    B = nodes.shape[0]
    nodes32 = nodes.astype(jnp.int32)
    nidx32 = neigh_idx.astype(jnp.int32)
    nidx_p = jnp.pad(nidx32, ((0, 0), (0, 128 - _K)))
    nodes_p = jnp.zeros((_BP,), jnp.int32).at[:B].set(nodes32)
    self_f, neigh_s = _sc_gather(nodes_p, nidx_p, emb)
    w1 = W[:, :_D]
    w2 = W[:, _D:] * (1.0 / _K)
    return _tc_combine(w1, w2, self_f, neigh_s, B)
